# Initial kernel scaffold; baseline (speedup 1.0000x reference)
import jax
import jax.numpy as jnp
from jax.experimental import pallas as pl


def kernel(x, Wk, bk, Wv, bv, Wq, bq, ln_w, ln_b, row_indices, col_indices):
    raise NotImplementedError("write your pallas kernel here")


# speedup vs baseline: 118.9028x; 118.9028x over previous
import functools

import jax
import jax.numpy as jnp
from jax import lax
from jax.experimental import pallas as pl
from jax.experimental.pallas import tpu as pltpu
from jax.experimental.pallas import tpu_sc as plsc

L = 2
N = 10000
INP = 128
KEY = 64
VAL = 64

N_PAD = 10240
TW = 144
CHUNK = 128

TC_BLK = 1000
XB = 2000


def _tca_body(x_ref, w_ref, b_ref, smax_ref):
    i = pl.program_id(0)
    y = jnp.dot(x_ref[...], w_ref[...], preferred_element_type=jnp.float32)
    y = y + b_ref[...]
    k = y[:, 0:KEY]
    q = y[:, 2 * KEY:3 * KEY]
    s = jnp.sum(k * q, axis=1, keepdims=True)
    m = jnp.max(s)

    @pl.when(i == 0)
    def _():
        smax_ref[0, 0] = m

    @pl.when(i > 0)
    def _():
        smax_ref[0, 0] = jnp.maximum(smax_ref[0, 0], m)


def _tca(x2, w, b):
    g = x2.shape[0] // XB
    return pl.pallas_call(
        _tca_body,
        grid=(g,),
        in_specs=[
            pl.BlockSpec((XB, INP), lambda i: (i, 0)),
            pl.BlockSpec((INP, 3 * KEY), lambda i: (0, 0)),
            pl.BlockSpec((1, 3 * KEY), lambda i: (0, 0)),
        ],
        out_specs=pl.BlockSpec(memory_space=pltpu.SMEM),
        out_shape=jax.ShapeDtypeStruct((1, 1), jnp.float32),
    )(x2, w, b)


def _tcb_body(x0_ref, x1_ref, w_ref, b_ref, smax_ref, tab_ref):
    w = w_ref[...]
    b = b_ref[...]
    smax = smax_ref[0, 0]

    def proj(x):
        y = jnp.dot(x, w, preferred_element_type=jnp.float32) + b
        k = y[:, 0:KEY]
        v = y[:, KEY:2 * KEY]
        q = y[:, 2 * KEY:3 * KEY]
        s = jnp.sum(k * q, axis=1, keepdims=True)
        es = jnp.exp(s - smax)
        return v * es, es

    tv0, es0 = proj(x0_ref[...])
    tv1, es1 = proj(x1_ref[...])
    tab_ref[:, 0:KEY] = tv0
    tab_ref[:, KEY:2 * KEY] = tv1
    lane = lax.broadcasted_iota(jnp.int32, (TC_BLK, TW - 2 * KEY), 1)
    esb = (jnp.where(lane == 0, es0, 0.0) + jnp.where(lane == 1, es1, 0.0))
    tab_ref[:, 2 * KEY:TW] = esb


def _tcb(x2, w, b, smax):
    g = N // TC_BLK
    return pl.pallas_call(
        _tcb_body,
        grid=(g,),
        in_specs=[
            pl.BlockSpec((TC_BLK, INP), lambda i: (i, 0)),
            pl.BlockSpec((TC_BLK, INP), lambda i: (i + N // TC_BLK, 0)),
            pl.BlockSpec((INP, 3 * KEY), lambda i: (0, 0)),
            pl.BlockSpec((1, 3 * KEY), lambda i: (0, 0)),
            pl.BlockSpec(memory_space=pltpu.SMEM),
        ],
        out_specs=pl.BlockSpec((TC_BLK, TW), lambda i: (i, 0)),
        out_shape=jax.ShapeDtypeStruct((N, TW), jnp.float32),
    )(x2, x2, w, b, smax)


def _sc_body(n_chunks, n_iters, rows_per_sub,
             tab_hbm, row_hbm, col_hbm, zero_hbm, acc_hbm,
             colv, rowv, rows_v, shared, sem):
    c = lax.axis_index("c")
    s = lax.axis_index("s")
    wid = s * 2 + c

    pltpu.sync_copy(zero_hbm.at[pl.ds(s * rows_per_sub, rows_per_sub)],
                    shared.at[pl.ds(s * rows_per_sub, rows_per_sub)])
    plsc.subcore_barrier()

    def body(i, carry):
        chunk = i * 32 + wid

        @pl.when(chunk < n_chunks)
        def _():
            base = pl.multiple_of(chunk * CHUNK, CHUNK)
            pltpu.sync_copy(col_hbm.at[pl.ds(base, CHUNK)], colv)
            pltpu.sync_copy(row_hbm.at[pl.ds(base, CHUNK)], rowv)
            pltpu.async_copy(tab_hbm.at[colv], rows_v, sem).wait()
            pltpu.sync_copy(rows_v, shared.at[rowv], add=True)

        return carry

    lax.fori_loop(0, n_iters, body, 0)
    plsc.subcore_barrier()

    pltpu.sync_copy(shared.at[pl.ds(s * rows_per_sub, rows_per_sub)],
                    acc_hbm.at[c].at[pl.ds(s * rows_per_sub, rows_per_sub)])


def _sc_edges(table, row_indices, col_indices):
    e = row_indices.shape[0]
    n_chunks = e // CHUNK
    n_iters = (n_chunks + 31) // 32
    rows_per_sub = N_PAD // 16
    zero = jnp.zeros((N_PAD, TW), jnp.float32)
    mesh = plsc.VectorSubcoreMesh(core_axis_name="c", subcore_axis_name="s")
    kern = pl.kernel(
        functools.partial(_sc_body, n_chunks, n_iters, rows_per_sub),
        out_type=jax.ShapeDtypeStruct((2, N_PAD, TW), jnp.float32),
        mesh=mesh,
        scratch_types=[
            pltpu.VMEM((CHUNK,), jnp.int32),
            pltpu.VMEM((CHUNK,), jnp.int32),
            pltpu.VMEM((CHUNK, TW), jnp.float32),
            pltpu.VMEM_SHARED((N_PAD, TW), jnp.float32),
            pltpu.SemaphoreType.DMA,
        ],
        compiler_params=pltpu.CompilerParams(use_tc_tiling_on_sc=False),
    )
    return kern(table, row_indices, col_indices, zero)


def _tcc_body(a0_ref, a1_ref, lnw_ref, lnb_ref, out_ref):
    a = a0_ref[0] + a1_ref[0]
    lnw = lnw_ref[...]
    lnb = lnb_ref[...]

    def finish(o, ws):
        ws = jnp.where(ws > 0.0, ws, 1.0)
        o = o / ws
        o = o * jax.nn.sigmoid(o)
        mean = jnp.mean(o, axis=1, keepdims=True)
        d = o - mean
        var = jnp.mean(d * d, axis=1, keepdims=True)
        return d * lax.rsqrt(var + 1e-5) * lnw + lnb

    out_ref[0] = finish(a[:, 0:KEY], a[:, 2 * KEY:2 * KEY + 1])
    out_ref[1] = finish(a[:, KEY:2 * KEY], a[:, 2 * KEY + 1:2 * KEY + 2])


def _tcc(acc, lnw, lnb):
    g = N // TC_BLK
    return pl.pallas_call(
        _tcc_body,
        grid=(g,),
        in_specs=[
            pl.BlockSpec((1, TC_BLK, TW), lambda i: (0, i, 0)),
            pl.BlockSpec((1, TC_BLK, TW), lambda i: (1, i, 0)),
            pl.BlockSpec((1, KEY), lambda i: (0, 0)),
            pl.BlockSpec((1, KEY), lambda i: (0, 0)),
        ],
        out_specs=pl.BlockSpec((L, TC_BLK, KEY), lambda i: (0, i, 0)),
        out_shape=jax.ShapeDtypeStruct((L, N, VAL), jnp.float32),
    )(acc, acc, lnw, lnb)


def kernel(x, Wk, bk, Wv, bv, Wq, bq, ln_w, ln_b, row_indices, col_indices):
    x2 = x.reshape(L * N, INP)
    w = jnp.concatenate([Wk, Wv, Wq], axis=0).T
    b = jnp.concatenate([bk, bv, bq]).reshape(1, 3 * KEY)

    smax = _tca(x2, w, b)
    table = _tcb(x2, w, b, smax)
    acc = _sc_edges(table, row_indices, col_indices)
    out = _tcc(acc, ln_w.reshape(1, VAL), ln_b.reshape(1, VAL))
    return (out, row_indices, col_indices)


# speedup vs baseline: 170.0163x; 1.4299x over previous
import functools

import jax
import jax.numpy as jnp
from jax import lax
from jax.experimental import pallas as pl
from jax.experimental.pallas import tpu as pltpu
from jax.experimental.pallas import tpu_sc as plsc

L = 2
N = 10000
INP = 128
KEY = 64
VAL = 64

N_PAD = 10240
TW = 144
CHUNK = 128

TC_BLK = 1000
XB = 2000


def _tca_body(x_ref, w_ref, b_ref, smax_ref):
    i = pl.program_id(0)
    y = jnp.dot(x_ref[...], w_ref[...], preferred_element_type=jnp.float32)
    y = y + b_ref[...]
    k = y[:, 0:KEY]
    q = y[:, 2 * KEY:3 * KEY]
    s = jnp.sum(k * q, axis=1, keepdims=True)
    m = jnp.max(s)

    @pl.when(i == 0)
    def _():
        smax_ref[0, 0] = m

    @pl.when(i > 0)
    def _():
        smax_ref[0, 0] = jnp.maximum(smax_ref[0, 0], m)


def _tca(x2, w, b):
    g = x2.shape[0] // XB
    return pl.pallas_call(
        _tca_body,
        grid=(g,),
        in_specs=[
            pl.BlockSpec((XB, INP), lambda i: (i, 0)),
            pl.BlockSpec((INP, 3 * KEY), lambda i: (0, 0)),
            pl.BlockSpec((1, 3 * KEY), lambda i: (0, 0)),
        ],
        out_specs=pl.BlockSpec(memory_space=pltpu.SMEM),
        out_shape=jax.ShapeDtypeStruct((1, 1), jnp.float32),
    )(x2, w, b)


def _tcb_body(x0_ref, x1_ref, w_ref, b_ref, smax_ref, tab_ref):
    w = w_ref[...]
    b = b_ref[...]
    smax = smax_ref[0, 0]

    def proj(x):
        y = jnp.dot(x, w, preferred_element_type=jnp.float32) + b
        k = y[:, 0:KEY]
        v = y[:, KEY:2 * KEY]
        q = y[:, 2 * KEY:3 * KEY]
        s = jnp.sum(k * q, axis=1, keepdims=True)
        es = jnp.exp(s - smax)
        return v * es, es

    tv0, es0 = proj(x0_ref[...])
    tv1, es1 = proj(x1_ref[...])
    tab_ref[:, 0:KEY] = tv0
    tab_ref[:, KEY:2 * KEY] = tv1
    lane = lax.broadcasted_iota(jnp.int32, (TC_BLK, TW - 2 * KEY), 1)
    esb = (jnp.where(lane == 0, es0, 0.0) + jnp.where(lane == 1, es1, 0.0))
    tab_ref[:, 2 * KEY:TW] = esb


def _tcb(x2, w, b, smax):
    g = N // TC_BLK
    return pl.pallas_call(
        _tcb_body,
        grid=(g,),
        in_specs=[
            pl.BlockSpec((TC_BLK, INP), lambda i: (i, 0)),
            pl.BlockSpec((TC_BLK, INP), lambda i: (i + N // TC_BLK, 0)),
            pl.BlockSpec((INP, 3 * KEY), lambda i: (0, 0)),
            pl.BlockSpec((1, 3 * KEY), lambda i: (0, 0)),
            pl.BlockSpec(memory_space=pltpu.SMEM),
        ],
        out_specs=pl.BlockSpec((TC_BLK, TW), lambda i: (i, 0)),
        out_shape=jax.ShapeDtypeStruct((N, TW), jnp.float32),
    )(x2, x2, w, b, smax)


NBUF = 2


def _sc_body(n_chunks, n_iters, rows_per_sub,
             tab_hbm, row_hbm, col_hbm, zero_hbm, acc_hbm,
             colv, rowv, bufs, shared, gsem, isem):
    c = lax.axis_index("c")
    s = lax.axis_index("s")
    wid = s * 2 + c

    def idx_load(j, b):
        base = pl.multiple_of((j * 32 + wid) * CHUNK, CHUNK)
        pltpu.async_copy(col_hbm.at[pl.ds(base, CHUNK)], colv.at[b], isem)
        pltpu.async_copy(row_hbm.at[pl.ds(base, CHUNK)], rowv.at[b], isem)

    def idx_wait(j, b):
        base = pl.multiple_of((j * 32 + wid) * CHUNK, CHUNK)
        pltpu.make_async_copy(col_hbm.at[pl.ds(base, CHUNK)], colv.at[b],
                              isem).wait()
        pltpu.make_async_copy(row_hbm.at[pl.ds(base, CHUNK)], rowv.at[b],
                              isem).wait()

    def gather(b):
        pltpu.async_copy(tab_hbm.at[colv.at[b]], bufs.at[b], gsem)

    def gather_wait(b):
        pltpu.make_async_copy(tab_hbm.at[colv.at[b]], bufs.at[b], gsem).wait()

    def valid(j):
        return (j * 32 + wid) < n_chunks

    pltpu.sync_copy(zero_hbm,
                    shared.at[pl.ds(s * rows_per_sub, rows_per_sub)])

    for b in range(NBUF):
        @pl.when(valid(b))
        def _():
            idx_load(b, b)
    plsc.subcore_barrier()

    @pl.when(valid(0))
    def _():
        idx_wait(0, 0)
        gather(0)

    def body(i, carry):
        for u in range(NBUF):
            j = i * NBUF + u
            nxt = j + 1
            bn = (u + 1) % NBUF

            @pl.when(valid(nxt))
            def _():
                idx_wait(nxt, bn)
                gather(bn)

            @pl.when(valid(j))
            def _():
                gather_wait(u)
                pltpu.sync_copy(bufs.at[u], shared.at[rowv.at[u]], add=True)

            pf = j + NBUF

            @pl.when(valid(pf))
            def _():
                idx_load(pf, u)

        return carry

    lax.fori_loop(0, (n_iters + NBUF - 1) // NBUF, body, 0)
    plsc.subcore_barrier()

    pltpu.sync_copy(shared.at[pl.ds(s * rows_per_sub, rows_per_sub)],
                    acc_hbm.at[c].at[pl.ds(s * rows_per_sub, rows_per_sub)])


def _sc_edges(table, row_indices, col_indices):
    e = row_indices.shape[0]
    n_chunks = e // CHUNK
    n_iters = (n_chunks + 31) // 32
    rows_per_sub = N_PAD // 16
    zero = jnp.zeros((rows_per_sub, TW), jnp.float32)
    mesh = plsc.VectorSubcoreMesh(core_axis_name="c", subcore_axis_name="s")
    kern = pl.kernel(
        functools.partial(_sc_body, n_chunks, n_iters, rows_per_sub),
        out_type=jax.ShapeDtypeStruct((2, N_PAD, TW), jnp.float32),
        mesh=mesh,
        scratch_types=[
            pltpu.VMEM((NBUF, CHUNK), jnp.int32),
            pltpu.VMEM((NBUF, CHUNK), jnp.int32),
            pltpu.VMEM((NBUF, CHUNK, TW), jnp.float32),
            pltpu.VMEM_SHARED((N_PAD, TW), jnp.float32),
            pltpu.SemaphoreType.DMA,
            pltpu.SemaphoreType.DMA,
        ],
        compiler_params=pltpu.CompilerParams(use_tc_tiling_on_sc=False),
    )
    return kern(table, row_indices, col_indices, zero)


def _tcc_body(a0_ref, a1_ref, lnw_ref, lnb_ref, out_ref):
    a = a0_ref[0] + a1_ref[0]
    lnw = lnw_ref[...]
    lnb = lnb_ref[...]

    def finish(o, ws):
        ws = jnp.where(ws > 0.0, ws, 1.0)
        o = o / ws
        o = o * jax.nn.sigmoid(o)
        mean = jnp.mean(o, axis=1, keepdims=True)
        d = o - mean
        var = jnp.mean(d * d, axis=1, keepdims=True)
        return d * lax.rsqrt(var + 1e-5) * lnw + lnb

    out_ref[0] = finish(a[:, 0:KEY], a[:, 2 * KEY:2 * KEY + 1])
    out_ref[1] = finish(a[:, KEY:2 * KEY], a[:, 2 * KEY + 1:2 * KEY + 2])


def _tcc(acc, lnw, lnb):
    g = N // TC_BLK
    return pl.pallas_call(
        _tcc_body,
        grid=(g,),
        in_specs=[
            pl.BlockSpec((1, TC_BLK, TW), lambda i: (0, i, 0)),
            pl.BlockSpec((1, TC_BLK, TW), lambda i: (1, i, 0)),
            pl.BlockSpec((1, KEY), lambda i: (0, 0)),
            pl.BlockSpec((1, KEY), lambda i: (0, 0)),
        ],
        out_specs=pl.BlockSpec((L, TC_BLK, KEY), lambda i: (0, i, 0)),
        out_shape=jax.ShapeDtypeStruct((L, N, VAL), jnp.float32),
    )(acc, acc, lnw, lnb)


def kernel(x, Wk, bk, Wv, bv, Wq, bq, ln_w, ln_b, row_indices, col_indices):
    x2 = x.reshape(L * N, INP)
    w = jnp.concatenate([Wk, Wv, Wq], axis=0).T
    b = jnp.concatenate([bk, bv, bq]).reshape(1, 3 * KEY)

    smax = _tca(x2, w, b)
    table = _tcb(x2, w, b, smax)
    acc = _sc_edges(table, row_indices, col_indices)
    out = _tcc(acc, ln_w.reshape(1, VAL), ln_b.reshape(1, VAL))
    return (out, row_indices, col_indices)
